# baseline (speedup 1.0000x reference)
import functools

import jax
import jax.numpy as jnp
from jax import lax
from jax.experimental import pallas as pl
from jax.experimental.pallas import tpu as pltpu
from jax.experimental.pallas import tpu_sc as plsc

_NUM_DAYS = 366
_B = 128
_D = 2 * 121 * 240
_C = 880
_K = _D // _C
_NW = 32
_ROWS_PER_W = _B // _NW
_CHUNKS_PER_W = _ROWS_PER_W * _K
_G = 16
_NGRP_FULL = _CHUNKS_PER_W // _G
_TAIL = _CHUNKS_PER_W - _NGRP_FULL * _G
_REG = 1e-09


def _sc_body(x0, x1, x2, x3, idx_hbm,
             w0, b0, w1, b1, w2, b2, w3, b3,
             o0, o1, o2, o3,
             idx_v, wv, bv, xv):
    c = lax.axis_index("c")
    s = lax.axis_index("s")
    wid = s * 2 + c
    pltpu.sync_copy(idx_hbm.at[wid], idx_v)
    row0 = wid * _CHUNKS_PER_W

    for (x_hbm, w_hbm, b_hbm, out_hbm, clip) in (
        (x0, w0, b0, o0, False),
        (x1, w1, b1, o1, True),
        (x2, w2, b2, o2, False),
        (x3, w3, b3, o3, True),
    ):
        def compute(nrows, clip=clip):
            def col_body(i, carry):
                off = pl.multiple_of(i * 16, 16)
                for r in range(nrows):
                    v = bv[r, pl.ds(off, 16)] + wv[r, pl.ds(off, 16)] * xv[r, pl.ds(off, 16)]
                    if clip:
                        v = jnp.maximum(v, _REG)
                    xv[r, pl.ds(off, 16)] = v
                return carry
            lax.fori_loop(0, _C // 16, col_body, 0)

        def group_body(g, carry, x_hbm=x_hbm, w_hbm=w_hbm, b_hbm=b_hbm,
                       out_hbm=out_hbm, clip=clip):
            r0 = row0 + g * _G
            pltpu.sync_copy(w_hbm.at[idx_v.at[g]], wv)
            pltpu.sync_copy(b_hbm.at[idx_v.at[g]], bv)
            pltpu.sync_copy(x_hbm.at[pl.ds(r0, _G)], xv)
            compute(_G)
            pltpu.sync_copy(xv, out_hbm.at[pl.ds(r0, _G)])
            return carry

        lax.fori_loop(0, _NGRP_FULL, group_body, 0)

        rt = row0 + _NGRP_FULL * _G
        pltpu.sync_copy(w_hbm.at[idx_v.at[_NGRP_FULL]], wv)
        pltpu.sync_copy(b_hbm.at[idx_v.at[_NGRP_FULL]], bv)
        pltpu.sync_copy(x_hbm.at[pl.ds(rt, _TAIL)], xv.at[pl.ds(0, _TAIL)])
        compute(_TAIL)
        pltpu.sync_copy(xv.at[pl.ds(0, _TAIL)], out_hbm.at[pl.ds(rt, _TAIL)])


_OUT_SHAPE = (_B * _K, _C)

_sc_kernel = functools.partial(
    pl.kernel,
    out_type=[jax.ShapeDtypeStruct(_OUT_SHAPE, jnp.float32)] * 4,
    mesh=plsc.VectorSubcoreMesh(core_axis_name="c", subcore_axis_name="s"),
    compiler_params=pltpu.CompilerParams(use_tc_tiling_on_sc=False),
    scratch_types=[
        pltpu.VMEM((_NGRP_FULL + 1, _G), jnp.int32),
        pltpu.VMEM((_G, _C), jnp.float32),
        pltpu.VMEM((_G, _C), jnp.float32),
        pltpu.VMEM((_G, _C), jnp.float32),
    ],
)(_sc_body)


def kernel(model_parameters_t2m_mu, model_parameters_t2m_sigma,
           model_parameters_tp_cube_root_mu, model_parameters_tp_cube_root_sigma,
           monthday,
           w_t2m_loc, b_t2m_loc, w_t2m_scale, b_t2m_scale,
           w_tp_loc, b_tp_loc, w_tp_scale, b_tp_scale):
    xs = [a.reshape(_B * _K, _C) for a in (
        model_parameters_t2m_mu, model_parameters_t2m_sigma,
        model_parameters_tp_cube_root_mu, model_parameters_tp_cube_root_sigma)]
    tabs = [a.reshape(_NUM_DAYS * _K, _C) for a in (
        w_t2m_loc, b_t2m_loc, w_t2m_scale, b_t2m_scale,
        w_tp_loc, b_tp_loc, w_tp_scale, b_tp_scale)]

    idx = (monthday[:, None] * _K + jnp.arange(_K, dtype=jnp.int32)[None, :])
    idx = idx.astype(jnp.int32).reshape(_NW, _CHUNKS_PER_W)
    pad = jnp.broadcast_to(idx[:, -1:], (_NW, _G - _TAIL))
    idx = jnp.concatenate([idx, pad], axis=1).reshape(_NW, _NGRP_FULL + 1, _G)

    outs = _sc_kernel(xs[0], xs[1], xs[2], xs[3], idx,
                      tabs[0], tabs[1], tabs[2], tabs[3],
                      tabs[4], tabs[5], tabs[6], tabs[7])

    shape = model_parameters_t2m_mu.shape
    return tuple(o.reshape(shape) for o in outs)


# speedup vs baseline: 1.0967x; 1.0967x over previous
import functools

import jax
import jax.numpy as jnp
from jax import lax
from jax.experimental import pallas as pl
from jax.experimental.pallas import tpu as pltpu
from jax.experimental.pallas import tpu_sc as plsc

_NUM_DAYS = 366
_B = 128
_D = 2 * 121 * 240
_CH = 5280
_NCH = _D // _CH
_NW = 32
_ROWS_PER_W = _B // _NW
_REG = 1e-09


def _sc_body(x0, x1, x2, x3, md_hbm,
             w0, b0, w1, b1, w2, b2, w3, b3,
             o0, o1, o2, o3,
             md_v, wrow, brow, xch):
    c = lax.axis_index("c")
    s = lax.axis_index("s")
    wid = s * 2 + c
    pltpu.sync_copy(md_hbm, md_v)

    for k in range(_ROWS_PER_W):
        b = wid * _ROWS_PER_W + k
        x_off = b * _D

        for (x_hbm, w_hbm, b_hbm, out_hbm, clip) in (
            (x0, w0, b0, o0, False),
            (x1, w1, b1, o1, True),
            (x2, w2, b2, o2, False),
            (x3, w3, b3, o3, True),
        ):
            pltpu.sync_copy(w_hbm.at[md_v.at[b]], wrow)
            pltpu.sync_copy(b_hbm.at[md_v.at[b]], brow)

            def chunk_body(j, carry, x_hbm=x_hbm, out_hbm=out_hbm,
                           clip=clip, x_off=x_off):
                cbase = pl.multiple_of(j * _CH, 8)
                pltpu.sync_copy(x_hbm.at[pl.ds(x_off + cbase, _CH)], xch)

                def col_body(i, carry2, cbase=cbase, clip=clip):
                    off = pl.multiple_of(i * 160, 16)
                    for u in range(10):
                        o = off + u * 16
                        v = (brow[0, pl.ds(cbase + o, 16)]
                             + wrow[0, pl.ds(cbase + o, 16)] * xch[pl.ds(o, 16)])
                        if clip:
                            v = jnp.maximum(v, _REG)
                        xch[pl.ds(o, 16)] = v
                    return carry2

                lax.fori_loop(0, _CH // 160, col_body, 0)
                pltpu.sync_copy(xch, out_hbm.at[pl.ds(x_off + cbase, _CH)])
                return carry

            lax.fori_loop(0, _NCH, chunk_body, 0)


_N1 = _B * _D

_sc_kernel = functools.partial(
    pl.kernel,
    out_type=[jax.ShapeDtypeStruct((_N1,), jnp.float32)] * 4,
    mesh=plsc.VectorSubcoreMesh(core_axis_name="c", subcore_axis_name="s"),
    compiler_params=pltpu.CompilerParams(use_tc_tiling_on_sc=False),
    scratch_types=[
        pltpu.VMEM((_B, 1), jnp.int32),
        pltpu.VMEM((1, _D), jnp.float32),
        pltpu.VMEM((1, _D), jnp.float32),
        pltpu.VMEM((_CH,), jnp.float32),
    ],
)(_sc_body)


def kernel(model_parameters_t2m_mu, model_parameters_t2m_sigma,
           model_parameters_tp_cube_root_mu, model_parameters_tp_cube_root_sigma,
           monthday,
           w_t2m_loc, b_t2m_loc, w_t2m_scale, b_t2m_scale,
           w_tp_loc, b_tp_loc, w_tp_scale, b_tp_scale):
    xs = [a.reshape(_N1) for a in (
        model_parameters_t2m_mu, model_parameters_t2m_sigma,
        model_parameters_tp_cube_root_mu, model_parameters_tp_cube_root_sigma)]
    tabs = [a.reshape(_NUM_DAYS, _D) for a in (
        w_t2m_loc, b_t2m_loc, w_t2m_scale, b_t2m_scale,
        w_tp_loc, b_tp_loc, w_tp_scale, b_tp_scale)]
    md = monthday.reshape(_B, 1).astype(jnp.int32)

    outs = _sc_kernel(xs[0], xs[1], xs[2], xs[3], md,
                      tabs[0], tabs[1], tabs[2], tabs[3],
                      tabs[4], tabs[5], tabs[6], tabs[7])

    shape = model_parameters_t2m_mu.shape
    return tuple(o.reshape(shape) for o in outs)


# speedup vs baseline: 2.9074x; 2.6512x over previous
import functools

import jax
import jax.numpy as jnp
from jax import lax
from jax.experimental import pallas as pl
from jax.experimental.pallas import tpu as pltpu
from jax.experimental.pallas import tpu_sc as plsc

_NUM_DAYS = 366
_B = 128
_P = (121, 240)
_NW = 32
_ROWS_PER_W = _B // _NW
_REG = 1e-09


def _sc_body(x0, x1, x2, x3, md_hbm,
             w0, b0, w1, b1, w2, b2, w3, b3,
             o0, o1, o2, o3,
             md_v, wpl, bpl, xpl):
    c = lax.axis_index("c")
    s = lax.axis_index("s")
    wid = s * 2 + c
    pltpu.sync_copy(md_hbm, md_v)

    for k in range(_ROWS_PER_W):
        b = wid * _ROWS_PER_W + k

        for ch in range(2):
            for (x_hbm, w_hbm, b_hbm, out_hbm, clip) in (
                (x0, w0, b0, o0, False),
                (x1, w1, b1, o1, True),
                (x2, w2, b2, o2, False),
                (x3, w3, b3, o3, True),
            ):
                vals = md_v[pl.ds(pl.multiple_of(b * 8, 8), 16)]
                d0 = vals[0]
                pltpu.sync_copy(w_hbm.at[pl.ds(d0, 1), ch], wpl)
                pltpu.sync_copy(b_hbm.at[pl.ds(d0, 1), ch], bpl)
                pltpu.sync_copy(x_hbm.at[pl.ds(b, 1), ch], xpl)

                def row_body(r, carry, clip=clip):
                    def col_body(i, carry2, r=r, clip=clip):
                        o = pl.multiple_of(i * 16, 16)
                        v = (bpl[0, r, pl.ds(o, 16)]
                             + wpl[0, r, pl.ds(o, 16)] * xpl[0, r, pl.ds(o, 16)])
                        if clip:
                            v = jnp.maximum(v, _REG)
                        xpl[0, r, pl.ds(o, 16)] = v
                        return carry2
                    lax.fori_loop(0, _P[1] // 16, col_body, 0)
                    return carry

                lax.fori_loop(0, _P[0], row_body, 0)
                pltpu.sync_copy(xpl, out_hbm.at[pl.ds(b, 1), ch])


_sc_kernel = functools.partial(
    pl.kernel,
    out_type=[jax.ShapeDtypeStruct((_B, 2) + _P, jnp.float32)] * 4,
    mesh=plsc.VectorSubcoreMesh(core_axis_name="c", subcore_axis_name="s"),
    compiler_params=pltpu.CompilerParams(use_tc_tiling_on_sc=True),
    scratch_types=[
        pltpu.VMEM((_B * 8 + 16,), jnp.int32),
        pltpu.VMEM((1,) + _P, jnp.float32),
        pltpu.VMEM((1,) + _P, jnp.float32),
        pltpu.VMEM((1,) + _P, jnp.float32),
    ],
)(_sc_body)


def kernel(model_parameters_t2m_mu, model_parameters_t2m_sigma,
           model_parameters_tp_cube_root_mu, model_parameters_tp_cube_root_sigma,
           monthday,
           w_t2m_loc, b_t2m_loc, w_t2m_scale, b_t2m_scale,
           w_tp_loc, b_tp_loc, w_tp_scale, b_tp_scale):
    xs = (model_parameters_t2m_mu, model_parameters_t2m_sigma,
          model_parameters_tp_cube_root_mu, model_parameters_tp_cube_root_sigma)
    tabs = [a for a in (
        w_t2m_loc, b_t2m_loc, w_t2m_scale, b_t2m_scale,
        w_tp_loc, b_tp_loc, w_tp_scale, b_tp_scale)]
    md = jnp.concatenate([jnp.repeat(monthday.astype(jnp.int32), 8),
                          jnp.zeros((16,), jnp.int32)])

    outs = _sc_kernel(xs[0], xs[1], xs[2], xs[3], md,
                      tabs[0], tabs[1], tabs[2], tabs[3],
                      tabs[4], tabs[5], tabs[6], tabs[7])
    return tuple(outs)


# speedup vs baseline: 3.9164x; 1.3470x over previous
import functools

import jax
import jax.numpy as jnp
from jax import lax
from jax.experimental import pallas as pl
from jax.experimental.pallas import tpu as pltpu
from jax.experimental.pallas import tpu_sc as plsc

_NUM_DAYS = 366
_B = 128
_P = (121, 240)
_NW = 32
_ROWS_PER_W = _B // _NW
_REG = 1e-09
_HALVES = ((0, 64), (64, 57))


def _sc_body(x0, x1, x2, x3, md_hbm,
             w0, b0, w1, b1, w2, b2, w3, b3,
             o0, o1, o2, o3,
             md_v, wp0, bp0, xp0, wp1, bp1, xp1,
             sw0, sb0, sx0, so0, sw1, sb1, sx1, so1):
    c = lax.axis_index("c")
    s = lax.axis_index("s")
    wid = s * 2 + c
    pltpu.sync_copy(md_hbm, md_v)

    keys = []
    for k in range(_ROWS_PER_W):
        b = wid * _ROWS_PER_W + k
        vals = md_v[pl.ds(pl.multiple_of(b * 8, 8), 16)]
        keys.append((b, vals[0]))

    stages = []
    for k in range(_ROWS_PER_W):
        for ch in range(2):
            for grp in ((x0, w0, b0, o0, False),
                        (x1, w1, b1, o1, True),
                        (x2, w2, b2, o2, False),
                        (x3, w3, b3, o3, True)):
                for half in _HALVES:
                    stages.append((k, ch, grp, half))

    bufs = ((wp0, bp0, xp0, sw0, sb0, sx0, so0),
            (wp1, bp1, xp1, sw1, sb1, sx1, so1))
    in_copies = [None, None]
    out_copies = [None, None]

    def issue(t):
        k, ch, (x_hbm, w_hbm, b_hbm, _, _), (r0, nr) = stages[t]
        b, d = keys[k]
        wpl, bpl, xpl, sw, sb, sx, _ = bufs[t % 2]
        if out_copies[t % 2] is not None:
            out_copies[t % 2].wait()
            out_copies[t % 2] = None
        cw = pltpu.async_copy(w_hbm.at[pl.ds(d, 1), ch, pl.ds(r0, nr)],
                              wpl, sw)
        cb = pltpu.async_copy(b_hbm.at[pl.ds(d, 1), ch, pl.ds(r0, nr)],
                              bpl, sb)
        cx = pltpu.async_copy(x_hbm.at[pl.ds(b, 1), ch, pl.ds(r0, nr)],
                              xpl, sx)
        in_copies[t % 2] = (cw, cb, cx)

    issue(0)
    n_stages = len(stages)
    for t in range(n_stages):
        if t + 1 < n_stages:
            issue(t + 1)
        for cpy in in_copies[t % 2]:
            cpy.wait()
        k, ch, (_, _, _, out_hbm, clip), (r0, nr) = stages[t]
        b, _ = keys[k]
        wpl, bpl, xpl, _, _, _, so = bufs[t % 2]

        def row_body(r, carry, clip=clip, wpl=wpl, bpl=bpl, xpl=xpl):
            for i in range(_P[1] // 16):
                o = i * 16
                v = (bpl[0, r, pl.ds(o, 16)]
                     + wpl[0, r, pl.ds(o, 16)] * xpl[0, r, pl.ds(o, 16)])
                if clip:
                    v = jnp.maximum(v, _REG)
                xpl[0, r, pl.ds(o, 16)] = v
            return carry

        lax.fori_loop(0, nr, row_body, 0)
        out_copies[t % 2] = pltpu.async_copy(
            xpl, out_hbm.at[pl.ds(b, 1), ch, pl.ds(r0, nr)], so)

    out_copies[0].wait()
    out_copies[1].wait()


_sc_kernel = functools.partial(
    pl.kernel,
    out_type=[jax.ShapeDtypeStruct((_B, 2) + _P, jnp.float32)] * 4,
    mesh=plsc.VectorSubcoreMesh(core_axis_name="c", subcore_axis_name="s"),
    compiler_params=pltpu.CompilerParams(use_tc_tiling_on_sc=True),
    scratch_types=[
        pltpu.VMEM((_B * 8 + 16,), jnp.int32),
        pltpu.VMEM((1, _HALVES[0][1], _P[1]), jnp.float32),
        pltpu.VMEM((1, _HALVES[0][1], _P[1]), jnp.float32),
        pltpu.VMEM((1, _HALVES[0][1], _P[1]), jnp.float32),
        pltpu.VMEM((1, _HALVES[1][1], _P[1]), jnp.float32),
        pltpu.VMEM((1, _HALVES[1][1], _P[1]), jnp.float32),
        pltpu.VMEM((1, _HALVES[1][1], _P[1]), jnp.float32),
        pltpu.SemaphoreType.DMA,
        pltpu.SemaphoreType.DMA,
        pltpu.SemaphoreType.DMA,
        pltpu.SemaphoreType.DMA,
        pltpu.SemaphoreType.DMA,
        pltpu.SemaphoreType.DMA,
        pltpu.SemaphoreType.DMA,
        pltpu.SemaphoreType.DMA,
    ],
)(_sc_body)


def kernel(model_parameters_t2m_mu, model_parameters_t2m_sigma,
           model_parameters_tp_cube_root_mu, model_parameters_tp_cube_root_sigma,
           monthday,
           w_t2m_loc, b_t2m_loc, w_t2m_scale, b_t2m_scale,
           w_tp_loc, b_tp_loc, w_tp_scale, b_tp_scale):
    md = jnp.concatenate([jnp.repeat(monthday.astype(jnp.int32), 8),
                          jnp.zeros((16,), jnp.int32)])

    outs = _sc_kernel(model_parameters_t2m_mu, model_parameters_t2m_sigma,
                      model_parameters_tp_cube_root_mu,
                      model_parameters_tp_cube_root_sigma, md,
                      w_t2m_loc, b_t2m_loc, w_t2m_scale, b_t2m_scale,
                      w_tp_loc, b_tp_loc, w_tp_scale, b_tp_scale)
    return tuple(outs)


# speedup vs baseline: 3.9170x; 1.0002x over previous
import functools

import jax
import jax.numpy as jnp
from jax import lax
from jax.experimental import pallas as pl
from jax.experimental.pallas import tpu as pltpu
from jax.experimental.pallas import tpu_sc as plsc

_NUM_DAYS = 366
_B = 128
_P = (121, 240)
_NW = 32
_ROWS_PER_W = _B // _NW
_REG = 1e-09
_CHUNKS = ((0, 64), (64, 57))
_DEPTH = len(_CHUNKS)


def _sc_body(x0, x1, x2, x3, md_hbm,
             w0, b0, w1, b1, w2, b2, w3, b3,
             o0, o1, o2, o3,
             md_v,
             wp0, bp0, xp0, wp1, bp1, xp1,
             sw0, sb0, sx0, so0, sw1, sb1, sx1, so1):
    c = lax.axis_index("c")
    s = lax.axis_index("s")
    wid = s * 2 + c
    pltpu.sync_copy(md_hbm, md_v)

    keys = []
    for k in range(_ROWS_PER_W):
        b = wid * _ROWS_PER_W + k
        vals = md_v[pl.ds(pl.multiple_of(b * 8, 8), 16)]
        keys.append((b, vals[0]))

    stages = []
    for k in range(_ROWS_PER_W):
        for ch in range(2):
            for grp in ((x0, w0, b0, o0, False),
                        (x1, w1, b1, o1, True),
                        (x2, w2, b2, o2, False),
                        (x3, w3, b3, o3, True)):
                for chunk in _CHUNKS:
                    stages.append((k, ch, grp, chunk))

    bufs = ((wp0, bp0, xp0, sw0, sb0, sx0, so0),
            (wp1, bp1, xp1, sw1, sb1, sx1, so1))
    in_copies = [None] * _DEPTH
    out_copies = [None] * _DEPTH

    def issue(t):
        k, ch, (x_hbm, w_hbm, b_hbm, _, _), (r0, nr) = stages[t]
        b, d = keys[k]
        wpl, bpl, xpl, sw, sb, sx, _ = bufs[t % _DEPTH]
        if out_copies[t % _DEPTH] is not None:
            out_copies[t % _DEPTH].wait()
            out_copies[t % _DEPTH] = None
        cw = pltpu.async_copy(w_hbm.at[pl.ds(d, 1), ch, pl.ds(r0, nr)],
                              wpl, sw)
        cb = pltpu.async_copy(b_hbm.at[pl.ds(d, 1), ch, pl.ds(r0, nr)],
                              bpl, sb)
        cx = pltpu.async_copy(x_hbm.at[pl.ds(b, 1), ch, pl.ds(r0, nr)],
                              xpl, sx)
        in_copies[t % _DEPTH] = (cw, cb, cx)

    n_stages = len(stages)
    for t in range(_DEPTH - 1):
        issue(t)
    for t in range(n_stages):
        if t + _DEPTH - 1 < n_stages:
            issue(t + _DEPTH - 1)
        for cpy in in_copies[t % _DEPTH]:
            cpy.wait()
        k, ch, (_, _, _, out_hbm, clip), (r0, nr) = stages[t]
        b, _ = keys[k]
        wpl, bpl, xpl, _, _, _, so = bufs[t % _DEPTH]

        def row_body(r, carry, clip=clip, wpl=wpl, bpl=bpl, xpl=xpl):
            for i in range(_P[1] // 16):
                o = i * 16
                v = (bpl[0, r, pl.ds(o, 16)]
                     + wpl[0, r, pl.ds(o, 16)] * xpl[0, r, pl.ds(o, 16)])
                if clip:
                    v = jnp.maximum(v, _REG)
                xpl[0, r, pl.ds(o, 16)] = v
            return carry

        lax.fori_loop(0, nr, row_body, 0)
        out_copies[t % _DEPTH] = pltpu.async_copy(
            xpl, out_hbm.at[pl.ds(b, 1), ch, pl.ds(r0, nr)], so)

    for oc in out_copies:
        oc.wait()


_sc_kernel = functools.partial(
    pl.kernel,
    out_type=[jax.ShapeDtypeStruct((_B, 2) + _P, jnp.float32)] * 4,
    mesh=plsc.VectorSubcoreMesh(core_axis_name="c", subcore_axis_name="s"),
    compiler_params=pltpu.CompilerParams(use_tc_tiling_on_sc=True),
    scratch_types=[
        pltpu.VMEM((_B * 8 + 16,), jnp.int32),
        pltpu.VMEM((1, _CHUNKS[0][1], _P[1]), jnp.float32),
        pltpu.VMEM((1, _CHUNKS[0][1], _P[1]), jnp.float32),
        pltpu.VMEM((1, _CHUNKS[0][1], _P[1]), jnp.float32),
        pltpu.VMEM((1, _CHUNKS[1][1], _P[1]), jnp.float32),
        pltpu.VMEM((1, _CHUNKS[1][1], _P[1]), jnp.float32),
        pltpu.VMEM((1, _CHUNKS[1][1], _P[1]), jnp.float32),
    ] + [pltpu.SemaphoreType.DMA] * 8,
)(_sc_body)


def kernel(model_parameters_t2m_mu, model_parameters_t2m_sigma,
           model_parameters_tp_cube_root_mu, model_parameters_tp_cube_root_sigma,
           monthday,
           w_t2m_loc, b_t2m_loc, w_t2m_scale, b_t2m_scale,
           w_tp_loc, b_tp_loc, w_tp_scale, b_tp_scale):
    md = jnp.concatenate([jnp.repeat(monthday.astype(jnp.int32), 8),
                          jnp.zeros((16,), jnp.int32)])

    outs = _sc_kernel(model_parameters_t2m_mu, model_parameters_t2m_sigma,
                      model_parameters_tp_cube_root_mu,
                      model_parameters_tp_cube_root_sigma, md,
                      w_t2m_loc, b_t2m_loc, w_t2m_scale, b_t2m_scale,
                      w_tp_loc, b_tp_loc, w_tp_scale, b_tp_scale)
    return tuple(outs)
